# baseline (device time: 144686 ns/iter reference)
import jax
import jax.numpy as jnp
from jax import lax
from jax.experimental import pallas as pl
from jax.experimental.pallas import tpu as pltpu


def kernel(x, dest):
    m, n = x.shape

    my_x = lax.axis_index("x")
    keep = dest == my_x
    k = jnp.sum(keep.astype(jnp.int32))
    s = m - k
    kpos = jnp.cumsum(keep.astype(jnp.int32)) - 1
    spos = jnp.cumsum((~keep).astype(jnp.int32)) - 1
    slot = jnp.where(keep, kpos, k + spos)
    src_of = jnp.zeros((m,), jnp.int32).at[slot].set(
        jnp.arange(m, dtype=jnp.int32), unique_indices=True
    )
    counts = jnp.stack([k, s]).astype(jnp.int32)

    def body(x_ref, src_ref, cnt_ref, out_ref, send_sem, recv_sem,
             local_sem):
        mx = lax.axis_index("x")
        peer = (1 - mx, lax.axis_index("y"), lax.axis_index("z"))
        kk = cnt_ref[0]
        ss = cnt_ref[1]
        rr = ss
        keep_base = jnp.where(mx == 0, 0, rr)
        remote_base = jnp.where(mx == 0, 0, m - ss)

        barrier_sem = pltpu.get_barrier_semaphore()
        pl.semaphore_signal(
            barrier_sem, inc=1, device_id=peer,
            device_id_type=pl.DeviceIdType.MESH,
        )
        pl.semaphore_wait(barrier_sem, 1)

        def row(ref, idx):
            return ref.at[pl.ds(pl.multiple_of(idx * n, n), n)]

        def rem_body(t, c):
            pltpu.make_async_remote_copy(
                src_ref=row(x_ref, src_ref[t]),
                dst_ref=row(out_ref, remote_base + (t - kk)),
                send_sem=send_sem,
                recv_sem=recv_sem,
                device_id=peer,
                device_id_type=pl.DeviceIdType.MESH,
            ).start()
            return c

        lax.fori_loop(kk, m, rem_body, jnp.int32(0))

        def loc_body(t, c):
            pltpu.make_async_copy(
                row(x_ref, src_ref[t]), row(out_ref, keep_base + t),
                local_sem,
            ).start()
            return c

        lax.fori_loop(0, kk, loc_body, jnp.int32(0))

        send_wait = pltpu.make_async_remote_copy(
            src_ref=row(x_ref, 0), dst_ref=row(out_ref, 0),
            send_sem=send_sem, recv_sem=recv_sem,
            device_id=peer, device_id_type=pl.DeviceIdType.MESH,
        )
        local_wait = pltpu.make_async_copy(
            row(x_ref, 0), row(out_ref, 0), local_sem
        )

        def drain(count, wait):
            def f8(i, c):
                for _ in range(8):
                    wait()
                return c

            def f1(i, c):
                wait()
                return c

            lax.fori_loop(0, count // 8, f8, jnp.int32(0))
            lax.fori_loop(0, count % 8, f1, jnp.int32(0))

        drain(ss, send_wait.wait_send)
        drain(kk, local_wait.wait)
        drain(rr, send_wait.wait_recv)

    out_flat = pl.pallas_call(
        body,
        out_shape=jax.ShapeDtypeStruct((m * n,), x.dtype),
        in_specs=[
            pl.BlockSpec(memory_space=pltpu.VMEM),
            pl.BlockSpec(memory_space=pltpu.SMEM),
            pl.BlockSpec(memory_space=pltpu.SMEM),
        ],
        out_specs=pl.BlockSpec(memory_space=pltpu.VMEM),
        scratch_shapes=[
            pltpu.SemaphoreType.DMA,
            pltpu.SemaphoreType.DMA,
            pltpu.SemaphoreType.DMA,
        ],
        compiler_params=pltpu.CompilerParams(collective_id=0),
    )(x.reshape(m * n), src_of, counts)
    return out_flat.reshape(m, n)


# device time: 144318 ns/iter; 1.0025x vs baseline; 1.0025x over previous
import jax
import jax.numpy as jnp
from jax import lax
from jax.experimental import pallas as pl
from jax.experimental.pallas import tpu as pltpu


def kernel(x, dest):
    m, n = x.shape

    my_x = lax.axis_index("x")
    keep = dest == my_x
    k = jnp.sum(keep.astype(jnp.int32))
    s = m - k
    csum = jnp.cumsum(keep.astype(jnp.int32))
    iota = jnp.arange(m, dtype=jnp.int32)
    slot = jnp.where(keep, csum - 1, k + iota - csum)
    src_of = jnp.zeros((m,), jnp.int32).at[slot].set(
        iota, unique_indices=True
    )
    counts = jnp.stack([k, s]).astype(jnp.int32)

    def body(x_ref, src_ref, cnt_ref, out_ref, send_sem, recv_sem,
             local_sem):
        mx = lax.axis_index("x")
        peer = (1 - mx, lax.axis_index("y"), lax.axis_index("z"))
        kk = cnt_ref[0]
        ss = cnt_ref[1]
        rr = ss
        keep_base = jnp.where(mx == 0, 0, rr)
        remote_base = jnp.where(mx == 0, 0, m - ss)

        barrier_sem = pltpu.get_barrier_semaphore()
        pl.semaphore_signal(
            barrier_sem, inc=1, device_id=peer,
            device_id_type=pl.DeviceIdType.MESH,
        )
        pl.semaphore_wait(barrier_sem, 1)

        def row(ref, idx):
            return ref.at[pl.ds(pl.multiple_of(idx * n, n), n)]

        rem_dst = remote_base - kk

        def rem1(t):
            pltpu.make_async_remote_copy(
                src_ref=row(x_ref, src_ref[t]),
                dst_ref=row(out_ref, rem_dst + t),
                send_sem=send_sem,
                recv_sem=recv_sem,
                device_id=peer,
                device_id_type=pl.DeviceIdType.MESH,
            ).start()

        def rem_body4(q, c):
            t = kk + q * 4
            for u in range(4):
                rem1(t + u)
            return c

        def rem_body1(t, c):
            rem1(t)
            return c

        lax.fori_loop(0, ss // 4, rem_body4, jnp.int32(0))
        lax.fori_loop(kk + (ss // 4) * 4, m, rem_body1, jnp.int32(0))

        def loc1(t):
            pltpu.make_async_copy(
                row(x_ref, src_ref[t]), row(out_ref, keep_base + t),
                local_sem,
            ).start()

        def loc_body4(q, c):
            t = q * 4
            for u in range(4):
                loc1(t + u)
            return c

        def loc_body1(t, c):
            loc1(t)
            return c

        lax.fori_loop(0, kk // 4, loc_body4, jnp.int32(0))
        lax.fori_loop((kk // 4) * 4, kk, loc_body1, jnp.int32(0))

        send_wait = pltpu.make_async_remote_copy(
            src_ref=row(x_ref, 0), dst_ref=row(out_ref, 0),
            send_sem=send_sem, recv_sem=recv_sem,
            device_id=peer, device_id_type=pl.DeviceIdType.MESH,
        )
        local_wait = pltpu.make_async_copy(
            row(x_ref, 0), row(out_ref, 0), local_sem
        )

        def drain(count, wait):
            def f8(i, c):
                for _ in range(8):
                    wait()
                return c

            def f1(i, c):
                wait()
                return c

            lax.fori_loop(0, count // 8, f8, jnp.int32(0))
            lax.fori_loop(0, count % 8, f1, jnp.int32(0))

        drain(ss, send_wait.wait_send)
        drain(kk, local_wait.wait)
        drain(rr, send_wait.wait_recv)

    out_flat = pl.pallas_call(
        body,
        out_shape=jax.ShapeDtypeStruct((m * n,), x.dtype),
        in_specs=[
            pl.BlockSpec(memory_space=pltpu.VMEM),
            pl.BlockSpec(memory_space=pltpu.SMEM),
            pl.BlockSpec(memory_space=pltpu.SMEM),
        ],
        out_specs=pl.BlockSpec(memory_space=pltpu.VMEM),
        scratch_shapes=[
            pltpu.SemaphoreType.DMA,
            pltpu.SemaphoreType.DMA,
            pltpu.SemaphoreType.DMA,
        ],
        compiler_params=pltpu.CompilerParams(collective_id=0),
    )(x.reshape(m * n), src_of, counts)
    return out_flat.reshape(m, n)


# device time: 144295 ns/iter; 1.0027x vs baseline; 1.0002x over previous
import jax
import jax.numpy as jnp
from jax import lax
from jax.experimental import pallas as pl
from jax.experimental.pallas import tpu as pltpu

CHUNK = 256


def kernel(x, dest):
    m, n = x.shape
    max_chunks = m // CHUNK

    my_x = lax.axis_index("x")
    keep = dest == my_x
    k = jnp.sum(keep.astype(jnp.int32))
    csum = jnp.cumsum(keep.astype(jnp.int32))
    iota = jnp.arange(m, dtype=jnp.int32)
    slot = jnp.where(keep, csum - 1, k + iota - csum)
    src_of = jnp.zeros((m,), jnp.int32).at[slot].set(iota, unique_indices=True)
    counts = jnp.stack([k, m - k]).astype(jnp.int32)

    def body(x_ref, src_ref, cnt_ref, out_ref, send_buf,
             send_sems, recv_sems):
        mx = lax.axis_index("x")
        peer = (1 - mx, lax.axis_index("y"), lax.axis_index("z"))
        kk = cnt_ref[0]
        ss = cnt_ref[1]
        rr = ss
        keep_base = jnp.where(mx == 0, 0, rr)
        remote_base = jnp.where(mx == 0, 0, m - ss)

        barrier_sem = pltpu.get_barrier_semaphore()
        pl.semaphore_signal(
            barrier_sem, inc=1, device_id=peer,
            device_id_type=pl.DeviceIdType.MESH,
        )
        pl.semaphore_wait(barrier_sem, 1)

        def rowslice(ref, idx, rows=1):
            return ref.at[pl.ds(pl.multiple_of(idx * n, n), rows * n)]

        def vcopy(dst_ref, dst_row, src_row):
            dst_ref[pl.ds(pl.multiple_of(dst_row * n, n), n)] = (
                x_ref[pl.ds(pl.multiple_of(src_row * n, n), n)]
            )

        n_send = (ss + CHUNK - 1) // CHUNK

        def send_chunk(j, off):
            pltpu.make_async_remote_copy(
                src_ref=rowslice(send_buf, off, CHUNK),
                dst_ref=rowslice(out_ref, remote_base + off, CHUNK),
                send_sem=send_sems.at[j],
                recv_sem=recv_sems.at[j],
                device_id=peer,
                device_id_type=pl.DeviceIdType.MESH,
            ).start()

        for c in range(max_chunks):
            def comp_body(t, z, c=c):
                vcopy(send_buf, t, src_ref[kk + t])
                return z

            lax.fori_loop(
                jnp.minimum(c * CHUNK, ss),
                jnp.minimum((c + 1) * CHUNK, ss),
                comp_body,
                jnp.int32(0),
            )

            @pl.when(c + 1 < n_send)
            def _(c=c):
                send_chunk(c, c * CHUNK)

        for c in range(max_chunks):
            @pl.when(c == n_send - 1)
            def _(c=c):
                send_chunk(c, ss - CHUNK)

        def keep_body(t, z):
            vcopy(out_ref, keep_base + t, src_ref[t])
            return z

        lax.fori_loop(0, kk, keep_body, jnp.int32(0))

        n_recv = (rr + CHUNK - 1) // CHUNK
        for j in range(max_chunks):
            @pl.when(j < n_recv)
            def _(j=j):
                pltpu.make_async_remote_copy(
                    src_ref=rowslice(send_buf, 0, CHUNK),
                    dst_ref=rowslice(out_ref, 0, CHUNK),
                    send_sem=send_sems.at[j],
                    recv_sem=recv_sems.at[j],
                    device_id=peer,
                    device_id_type=pl.DeviceIdType.MESH,
                ).wait_recv()

        for j in range(max_chunks):
            @pl.when(j < n_send)
            def _(j=j):
                pltpu.make_async_remote_copy(
                    src_ref=rowslice(send_buf, 0, CHUNK),
                    dst_ref=rowslice(out_ref, 0, CHUNK),
                    send_sem=send_sems.at[j],
                    recv_sem=recv_sems.at[j],
                    device_id=peer,
                    device_id_type=pl.DeviceIdType.MESH,
                ).wait_send()

    out_flat = pl.pallas_call(
        body,
        out_shape=jax.ShapeDtypeStruct((m * n,), x.dtype),
        in_specs=[
            pl.BlockSpec(memory_space=pltpu.VMEM),
            pl.BlockSpec(memory_space=pltpu.SMEM),
            pl.BlockSpec(memory_space=pltpu.SMEM),
        ],
        out_specs=pl.BlockSpec(memory_space=pltpu.VMEM),
        scratch_shapes=[
            pltpu.VMEM((m * n,), x.dtype),
            pltpu.SemaphoreType.DMA((m // CHUNK,)),
            pltpu.SemaphoreType.DMA((m // CHUNK,)),
        ],
        compiler_params=pltpu.CompilerParams(collective_id=0),
    )(x.reshape(m * n), src_of, counts)
    return out_flat.reshape(m, n)
